# Initial kernel scaffold; baseline (speedup 1.0000x reference)
import jax
import jax.numpy as jnp
from jax.experimental import pallas as pl


def kernel(indices, weight):
    raise NotImplementedError("write your pallas kernel here")


# speedup vs baseline: 1.1990x; 1.1990x over previous
import jax
import jax.numpy as jnp
from jax.experimental import pallas as pl
from jax.experimental.pallas import tpu as pltpu
from jax.experimental.pallas import tpu_sc as plsc

_GATHER_WINDOW = 128


def _sc_gather(idx_flat, weight, num_indices, embed_dim):
    mesh = plsc.VectorSubcoreMesh(
        core_axis_name="core", subcore_axis_name="subcore"
    )

    @pl.kernel(
        out_type=jax.ShapeDtypeStruct((num_indices, embed_dim), weight.dtype),
        mesh=mesh,
    )
    def gather_kernel(x_hbm, i_hbm, o_hbm):
        def body(i_vmem, o_vmem):
            pltpu.sync_copy(x_hbm.at[i_vmem.at[0]], o_vmem)

        pltpu.emit_pipeline(
            body,
            grid=(num_indices // _GATHER_WINDOW,),
            in_specs=[
                pl.BlockSpec((1, _GATHER_WINDOW), index_map=lambda i: (0, i))
            ],
            out_specs=[
                pl.BlockSpec(
                    (_GATHER_WINDOW, embed_dim), index_map=lambda i: (i, 0)
                )
            ],
            core_axis_name=("core", "subcore"),
            dimension_semantics=(pltpu.PARALLEL,),
        )(i_hbm, o_hbm)

    return gather_kernel(weight, idx_flat)


def kernel(indices, weight):
    batch, fields = indices.shape
    vocab, embed_dim = weight.shape
    num_indices = batch * fields
    idx_flat = indices.reshape(1, num_indices).astype(jnp.int32)
    out = _sc_gather(idx_flat, weight, num_indices, embed_dim)
    return out.reshape(batch, fields, embed_dim)


# speedup vs baseline: 1.7091x; 1.4255x over previous
import jax
import jax.numpy as jnp
from jax import lax
from jax.experimental import pallas as pl
from jax.experimental.pallas import tpu as pltpu
from jax.experimental.pallas import tpu_sc as plsc

_NB = 4


def _sc_gather(idx1d, weight, batch, fields, embed_dim):
    mesh = plsc.VectorSubcoreMesh(
        core_axis_name="core", subcore_axis_name="subcore"
    )
    info = plsc.get_sparse_core_info()
    nw = info.num_cores * info.num_subcores
    window = _NB * fields
    b_per_w = batch // nw
    steps = b_per_w // _NB
    idx_per_w = b_per_w * fields

    @pl.kernel(
        out_type=jax.ShapeDtypeStruct(
            (batch, fields, embed_dim), weight.dtype
        ),
        mesh=mesh,
        scratch_types=[
            pltpu.VMEM((idx_per_w,), jnp.int32),
            pltpu.VMEM((window, embed_dim), jnp.float32),
            pltpu.SemaphoreType.DMA,
        ],
    )
    def gather_kernel(x_hbm, i_hbm, o_hbm, idx_v, rows_v, sem):
        c = lax.axis_index("core")
        s = lax.axis_index("subcore")
        wid = s * info.num_cores + c
        pltpu.sync_copy(i_hbm.at[pl.ds(wid * idx_per_w, idx_per_w)], idx_v)
        b_base = wid * b_per_w

        @pl.loop(0, steps)
        def _(step):
            off = pl.multiple_of(step * window, 8)
            pltpu.sync_copy(
                x_hbm.at[idx_v.at[pl.ds(off, window)]], rows_v
            )
            pltpu.sync_copy(
                rows_v.reshape(_NB, fields, embed_dim),
                o_hbm.at[pl.ds(b_base + step * _NB, _NB)],
            )

    return gather_kernel(weight, idx1d)


def kernel(indices, weight):
    batch, fields = indices.shape
    vocab, embed_dim = weight.shape
    idx1d = indices.reshape(batch * fields).astype(jnp.int32)
    return _sc_gather(idx1d, weight, batch, fields, embed_dim)
